# baseline (speedup 1.0000x reference)
import functools

import jax
import jax.numpy as jnp
from jax import lax
from jax.experimental import pallas as pl
from jax.experimental.pallas import tpu as pltpu
from jax.experimental.pallas import tpu_sc as plsc

N = 10000
E = 320000
D = 128
H = 128
G = 64

NC = 2
NS = 16
NW = NC * NS
CB = 96
NCH = 108
EPW = NCH * CB
EP = EPW * NW
NP = 10240
RPT = NP // NS
CPT = NP // NS
NB = NP // 128


def _sc_counts(dst_hbm, out_hbm, idx_v, val_v, acc):
    cid = lax.axis_index("c")
    sid = lax.axis_index("s")
    wid = sid * NC + cid

    def zb(i, carry):
        val_v[pl.ds(i * 16, 16)] = jnp.zeros((16,), jnp.float32)
        return carry
    lax.fori_loop(0, CPT // 16, zb, 0)
    pltpu.sync_copy(val_v, acc.at[pl.ds(sid * CPT, CPT)])

    def ob(i, carry):
        val_v[pl.ds(i * 16, 16)] = jnp.ones((16,), jnp.float32)
        return carry
    lax.fori_loop(0, CB // 16, ob, 0)

    pltpu.sync_copy(dst_hbm.at[wid], idx_v)
    plsc.subcore_barrier()

    def body(j, carry):
        pltpu.sync_copy(val_v.at[pl.ds(0, CB)], acc.at[idx_v.at[j]], add=True)
        return carry
    lax.fori_loop(0, NCH, body, 0)

    plsc.subcore_barrier()
    pltpu.sync_copy(acc.at[pl.ds(sid * CPT, CPT)],
                    out_hbm.at[cid, pl.ds(sid * CPT, CPT)])


def _sc_scatter(u_hbm, src_hbm, dst_hbm, out_hbm, src_v, dst_v, row0, row1,
                row2, gs0, gs1, gs2, ss0, ss1, ss2, isem, ds0, ds1, zsem, acc):
    rows = [row0, row1, row2]
    gsem = [gs0, gs1, gs2]
    ssem = [ss0, ss1, ss2]
    dsem = [ds0, ds1]
    cid = lax.axis_index("c")
    sid = lax.axis_index("s")
    wid = sid * NC + cid

    pltpu.sync_copy(src_hbm.at[wid, 0], src_v.at[0])
    pltpu.sync_copy(src_hbm.at[wid, 1], src_v.at[1])
    pltpu.sync_copy(dst_hbm.at[wid, 0], dst_v.at[0])
    pltpu.sync_copy(dst_hbm.at[wid, 1], dst_v.at[1])
    pltpu.async_copy(u_hbm.at[src_v.at[0]], rows[0], gsem[0])

    def zb(i, carry):
        rows[2][i // 8, pl.ds((i % 8) * 16, 16)] = jnp.zeros((16,), jnp.float32)
        return carry
    lax.fori_loop(0, 40 * 8, zb, 0)
    base = sid * RPT
    for k in range(RPT // 40):
        pltpu.async_copy(rows[2].at[pl.ds(0, 40)],
                         acc.at[pl.ds(base + k * 40, 40)], zsem)
    for k in range(RPT // 40):
        pltpu.make_async_copy(rows[2].at[pl.ds(0, 40)],
                              acc.at[pl.ds(base + k * 40, 40)], zsem).wait()
    plsc.subcore_barrier()

    def chunk(j, k):
        s, s1, s2 = k % 3, (k + 1) % 3, (k + 2) % 3
        d, d2 = k % 4, (k + 2) % 4

        @pl.when(j >= 2)
        def _wait_scatter_jm2():
            pltpu.make_async_copy(rows[s1], acc.at[dst_v.at[(d + 2) % 4]],
                                  ssem[s1]).wait()

        @pl.when(j + 1 < NCH)
        def _issue_next_gather():
            @pl.when(j >= 1)
            def _wait_src_load():
                pltpu.make_async_copy(src_hbm.at[wid, j + 1], src_v.at[s1],
                                      isem).wait()
            pltpu.async_copy(u_hbm.at[src_v.at[s1]], rows[s1], gsem[s1])

        @pl.when(j >= 2)
        def _wait_dst_load():
            pltpu.make_async_copy(dst_hbm.at[wid, j], dst_v.at[d],
                                  dsem[k % 2]).wait()
        pltpu.make_async_copy(u_hbm.at[src_v.at[s]], rows[s], gsem[s]).wait()
        pltpu.async_copy(rows[s], acc.at[dst_v.at[d]], ssem[s], add=True)

        @pl.when(j + 2 < NCH)
        def _issue_next_loads():
            pltpu.async_copy(src_hbm.at[wid, j + 2], src_v.at[s2], isem)
            pltpu.async_copy(dst_hbm.at[wid, j + 2], dst_v.at[d2],
                             dsem[k % 2])

    def body(i, carry):
        j0 = 12 * i
        for k in range(12):
            chunk(j0 + k, k)
        return carry
    lax.fori_loop(0, NCH // 12, body, 0)
    pltpu.make_async_copy(rows[(NCH - 2) % 3],
                          acc.at[dst_v.at[(NCH - 2) % 4]],
                          ssem[(NCH - 2) % 3]).wait()
    pltpu.make_async_copy(rows[(NCH - 1) % 3],
                          acc.at[dst_v.at[(NCH - 1) % 4]],
                          ssem[(NCH - 1) % 3]).wait()

    plsc.subcore_barrier()
    pltpu.sync_copy(acc.at[pl.ds(base, RPT)],
                    out_hbm.at[cid, pl.ds(base, RPT)])


@functools.cache
def _sc_kernels():
    mesh = plsc.VectorSubcoreMesh(core_axis_name="c", subcore_axis_name="s",
                                  num_cores=NC, num_subcores=NS)
    counts = pl.kernel(
        _sc_counts,
        out_type=jax.ShapeDtypeStruct((NC, NP), jnp.float32),
        mesh=mesh,
        scratch_types=[
            pltpu.VMEM((NCH, CB), jnp.int32),
            pltpu.VMEM((CPT,), jnp.float32),
            pltpu.VMEM_SHARED((NP,), jnp.float32),
        ],
    )
    scatter = pl.kernel(
        _sc_scatter,
        out_type=jax.ShapeDtypeStruct((NC, NP, H), jnp.float32),
        mesh=mesh,
        scratch_types=[
            pltpu.VMEM((3, CB), jnp.int32),
            pltpu.VMEM((4, CB), jnp.int32),
            pltpu.VMEM((CB, H), jnp.float32),
            pltpu.VMEM((CB, H), jnp.float32),
            pltpu.VMEM((CB, H), jnp.float32),
            pltpu.SemaphoreType.DMA,
            pltpu.SemaphoreType.DMA,
            pltpu.SemaphoreType.DMA,
            pltpu.SemaphoreType.DMA,
            pltpu.SemaphoreType.DMA,
            pltpu.SemaphoreType.DMA,
            pltpu.SemaphoreType.DMA,
            pltpu.SemaphoreType.DMA,
            pltpu.SemaphoreType.DMA,
            pltpu.SemaphoreType.DMA,
            pltpu.VMEM_SHARED((NP, H), jnp.float32),
        ],
    )
    return counts, scatter


_P = lax.Precision.HIGHEST
NRB = 1024
NGB = NP // NRB


def _tc_mm0_body(x_ref, w_ref, o_ref):
    o_ref[...] = jnp.dot(x_ref[...], w_ref[...], precision=_P,
                         preferred_element_type=jnp.float32)


_tc_mm0 = pl.pallas_call(
    _tc_mm0_body,
    grid=(NGB,),
    in_specs=[pl.BlockSpec((NRB, D), lambda i: (i, 0)),
              pl.BlockSpec((D, H), lambda i: (0, 0))],
    out_specs=pl.BlockSpec((NRB, H), lambda i: (i, 0)),
    out_shape=jax.ShapeDtypeStruct((N, H), jnp.float32),
)


def _tc_scale_body(c_ref, hw_ref, u_ref, dinvb_ref):
    d2 = lax.rsqrt(1.0 + c_ref[0] + c_ref[1])
    i0 = lax.broadcasted_iota(jnp.int32, (128, 128), 0)
    i1 = lax.broadcasted_iota(jnp.int32, (128, 128), 1)
    eye = (i0 == i1).astype(jnp.float32)
    dm = d2[:, :, None] * eye[None, :, :]
    ones = jnp.ones((128, 128), jnp.float32)
    m = lax.dot_general(dm, ones, (((2,), (0,)), ((), ())),
                        precision=_P, preferred_element_type=jnp.float32)
    dinvb = jnp.reshape(m, (NRB, 128))
    u_ref[...] = hw_ref[...] * dinvb
    dinvb_ref[...] = dinvb


_tc_scale = pl.pallas_call(
    _tc_scale_body,
    grid=(NGB,),
    in_specs=[pl.BlockSpec((NC, NRB // 128, 128), lambda i: (0, i, 0)),
              pl.BlockSpec((NRB, H), lambda i: (i, 0))],
    out_specs=[pl.BlockSpec((NRB, H), lambda i: (i, 0))] * 2,
    out_shape=[jax.ShapeDtypeStruct((N, H), jnp.float32),
               jax.ShapeDtypeStruct((N, H), jnp.float32)],
)


def _tc_mid_body(p_ref, u_ref, dinvb_ref, b_ref, w_ref, out_ref):
    s = p_ref[0] + p_ref[1] + u_ref[...]
    h = jnp.maximum(s * dinvb_ref[...] + b_ref[...], 0.0)
    out_ref[...] = jnp.dot(h, w_ref[...], precision=_P,
                           preferred_element_type=jnp.float32) * dinvb_ref[...]


_tc_mid = pl.pallas_call(
    _tc_mid_body,
    grid=(NGB,),
    in_specs=[pl.BlockSpec((NC, NRB, H), lambda i: (0, i, 0)),
              pl.BlockSpec((NRB, H), lambda i: (i, 0)),
              pl.BlockSpec((NRB, H), lambda i: (i, 0)),
              pl.BlockSpec((1, H), lambda i: (0, 0)),
              pl.BlockSpec((H, H), lambda i: (0, 0))],
    out_specs=pl.BlockSpec((NRB, H), lambda i: (i, 0)),
    out_shape=jax.ShapeDtypeStruct((N, H), jnp.float32),
)


def _tc_fin_body(p_ref, u_ref, dinvb_ref, b_ref, batch_ref, out_ref, cnt_scr):
    i = pl.program_id(0)
    s = p_ref[0] + p_ref[1] + u_ref[...]
    h = jnp.maximum(s * dinvb_ref[...] + b_ref[...], 0.0)
    rv = lax.broadcasted_iota(jnp.int32, (NRB, H), 0) + i * NRB
    h = jnp.where(rv < N, h, 0.0)
    gi = lax.broadcasted_iota(jnp.int32, (G, NRB), 0)
    cv = lax.broadcasted_iota(jnp.int32, (G, NRB), 1) + i * NRB
    pmat = ((batch_ref[...] == gi) & (cv < N)).astype(jnp.float32)
    sums = jnp.dot(pmat, h, precision=_P,
                   preferred_element_type=jnp.float32)
    cntb = jnp.dot(pmat, jnp.ones((NRB, H), jnp.float32), precision=_P,
                   preferred_element_type=jnp.float32)

    @pl.when(i == 0)
    def _init():
        out_ref[...] = sums
        cnt_scr[...] = cntb

    @pl.when(i > 0)
    def _accum():
        out_ref[...] += sums
        cnt_scr[...] += cntb

    @pl.when(i == NGB - 1)
    def _finish():
        out_ref[...] = out_ref[...] / jnp.maximum(cnt_scr[...], 1.0)


_tc_fin = pl.pallas_call(
    _tc_fin_body,
    grid=(NGB,),
    in_specs=[pl.BlockSpec((NC, NRB, H), lambda i: (0, i, 0)),
              pl.BlockSpec((NRB, H), lambda i: (i, 0)),
              pl.BlockSpec((NRB, H), lambda i: (i, 0)),
              pl.BlockSpec((1, H), lambda i: (0, 0)),
              pl.BlockSpec((1, NRB), lambda i: (0, i))],
    out_specs=pl.BlockSpec((G, H), lambda i: (0, 0)),
    out_shape=jax.ShapeDtypeStruct((G, H), jnp.float32),
    scratch_shapes=[pltpu.VMEM((G, H), jnp.float32)],
)


def kernel(x, edge_index, batch, W0, b0, W1, b1, W2, b2):
    pad = EP - E
    pad_src = (jnp.arange(pad, dtype=jnp.int32) * 13) % N
    pad_dst = N + jnp.arange(pad, dtype=jnp.int32) % (NP - N)
    src_r = jnp.concatenate([edge_index[0], pad_src]).reshape(NW, NCH, CB)
    dst_r = jnp.concatenate([edge_index[1], pad_dst]).reshape(NW, NCH, CB)
    batch2 = batch.reshape(1, N)
    sc_counts, sc_scatter = _sc_kernels()

    hw0 = _tc_mm0(x, W0)
    cpart = sc_counts(dst_r)
    c3 = cpart.reshape(NC, NB, 128)
    u0, dinvb = _tc_scale(c3, hw0)

    p = sc_scatter(u0, src_r, dst_r)
    u1 = _tc_mid(p, u0, dinvb, b0.reshape(1, H), W1)
    p = sc_scatter(u1, src_r, dst_r)
    u2 = _tc_mid(p, u1, dinvb, b1.reshape(1, H), W2)
    p = sc_scatter(u2, src_r, dst_r)
    return _tc_fin(p, u2, dinvb, b2.reshape(1, H), batch2)


# speedup vs baseline: 1.0890x; 1.0890x over previous
import functools

import jax
import jax.numpy as jnp
from jax import lax
from jax.experimental import pallas as pl
from jax.experimental.pallas import tpu as pltpu
from jax.experimental.pallas import tpu_sc as plsc

N = 10000
E = 320000
D = 128
H = 128
G = 64

NC = 2
NS = 16
NW = NC * NS
CB = 128
NCH = 80
EPW = NCH * CB
EP = EPW * NW
NP = 10240
RPT = NP // NS
CPT = NP // NS
NB = NP // 128


def _sc_counts(dst_hbm, out_hbm, idx_v, val_v, acc):
    cid = lax.axis_index("c")
    sid = lax.axis_index("s")
    wid = sid * NC + cid

    def zb(i, carry):
        val_v[pl.ds(i * 16, 16)] = jnp.zeros((16,), jnp.float32)
        return carry
    lax.fori_loop(0, CPT // 16, zb, 0)
    pltpu.sync_copy(val_v, acc.at[pl.ds(sid * CPT, CPT)])

    def ob(i, carry):
        val_v[pl.ds(i * 16, 16)] = jnp.ones((16,), jnp.float32)
        return carry
    lax.fori_loop(0, CB // 16, ob, 0)

    pltpu.sync_copy(dst_hbm.at[wid], idx_v)
    plsc.subcore_barrier()

    def body(j, carry):
        pltpu.sync_copy(val_v.at[pl.ds(0, CB)], acc.at[idx_v.at[j]], add=True)
        return carry
    lax.fori_loop(0, NCH, body, 0)

    plsc.subcore_barrier()
    pltpu.sync_copy(acc.at[pl.ds(sid * CPT, CPT)],
                    out_hbm.at[cid, pl.ds(sid * CPT, CPT)])


def _sc_scatter(u_hbm, src_hbm, dst_hbm, out_hbm, src_v, dst_v, row_a, row_b,
                gsem_a, gsem_b, ssem_a, ssem_b, isem, zsem, acc):
    cid = lax.axis_index("c")
    sid = lax.axis_index("s")
    wid = sid * NC + cid

    pltpu.sync_copy(src_hbm.at[wid, 0], src_v.at[0])
    pltpu.sync_copy(dst_hbm.at[wid], dst_v)
    pltpu.async_copy(u_hbm.at[src_v.at[0]], row_a, gsem_a)
    pltpu.async_copy(src_hbm.at[wid, 1], src_v.at[1], isem)

    def zb(i, carry):
        row_b[i // 8, pl.ds((i % 8) * 16, 16)] = jnp.zeros((16,), jnp.float32)
        return carry
    lax.fori_loop(0, 40 * 8, zb, 0)
    base = sid * RPT
    for k in range(RPT // 40):
        pltpu.async_copy(row_b.at[pl.ds(0, 40)],
                         acc.at[pl.ds(base + k * 40, 40)], zsem)
    for k in range(RPT // 40):
        pltpu.make_async_copy(row_b.at[pl.ds(0, 40)],
                              acc.at[pl.ds(base + k * 40, 40)], zsem).wait()
    plsc.subcore_barrier()

    def half(j, sx_slot, rx, gx, sx, ry, gy, sy):
        @pl.when(j > 0)
        def _wait_prev_scatter():
            pltpu.make_async_copy(ry, acc.at[dst_v.at[j - 1]], sy).wait()

        @pl.when(j + 1 < NCH)
        def _next_gather():
            pltpu.make_async_copy(src_hbm.at[wid, j + 1],
                                  src_v.at[1 - sx_slot], isem).wait()
            pltpu.async_copy(u_hbm.at[src_v.at[1 - sx_slot]], ry, gy)

        pltpu.make_async_copy(u_hbm.at[src_v.at[sx_slot]], rx, gx).wait()
        pltpu.async_copy(rx, acc.at[dst_v.at[j]], sx, add=True)

        @pl.when(j + 2 < NCH)
        def _next_src_load():
            pltpu.async_copy(src_hbm.at[wid, j + 2], src_v.at[sx_slot], isem)

    def body(i, carry):
        half(2 * i, 0, row_a, gsem_a, ssem_a, row_b, gsem_b, ssem_b)
        half(2 * i + 1, 1, row_b, gsem_b, ssem_b, row_a, gsem_a, ssem_a)
        return carry
    lax.fori_loop(0, NCH // 2, body, 0)
    pltpu.make_async_copy(row_b, acc.at[dst_v.at[NCH - 1]], ssem_b).wait()

    plsc.subcore_barrier()
    pltpu.sync_copy(acc.at[pl.ds(base, RPT)],
                    out_hbm.at[cid, pl.ds(base, RPT)])


@functools.cache
def _sc_kernels():
    mesh = plsc.VectorSubcoreMesh(core_axis_name="c", subcore_axis_name="s",
                                  num_cores=NC, num_subcores=NS)
    counts = pl.kernel(
        _sc_counts,
        out_type=jax.ShapeDtypeStruct((NC, NP), jnp.float32),
        mesh=mesh,
        scratch_types=[
            pltpu.VMEM((NCH, CB), jnp.int32),
            pltpu.VMEM((CPT,), jnp.float32),
            pltpu.VMEM_SHARED((NP,), jnp.float32),
        ],
    )
    scatter = pl.kernel(
        _sc_scatter,
        out_type=jax.ShapeDtypeStruct((NC, NP, H), jnp.float32),
        mesh=mesh,
        scratch_types=[
            pltpu.VMEM((2, CB), jnp.int32),
            pltpu.VMEM((NCH, CB), jnp.int32),
            pltpu.VMEM((CB, H), jnp.float32),
            pltpu.VMEM((CB, H), jnp.float32),
            pltpu.SemaphoreType.DMA,
            pltpu.SemaphoreType.DMA,
            pltpu.SemaphoreType.DMA,
            pltpu.SemaphoreType.DMA,
            pltpu.SemaphoreType.DMA,
            pltpu.SemaphoreType.DMA,
            pltpu.VMEM_SHARED((NP, H), jnp.float32),
        ],
    )
    return counts, scatter


_P = lax.Precision.HIGHEST
NRB = 1024
NGB = NP // NRB


def _tc_mm0_body(x_ref, w_ref, o_ref):
    o_ref[...] = jnp.dot(x_ref[...], w_ref[...], precision=_P,
                         preferred_element_type=jnp.float32)


_tc_mm0 = pl.pallas_call(
    _tc_mm0_body,
    grid=(NGB,),
    in_specs=[pl.BlockSpec((NRB, D), lambda i: (i, 0)),
              pl.BlockSpec((D, H), lambda i: (0, 0))],
    out_specs=pl.BlockSpec((NRB, H), lambda i: (i, 0)),
    out_shape=jax.ShapeDtypeStruct((N, H), jnp.float32),
)


def _tc_scale_body(c_ref, hw_ref, u_ref, dinvb_ref):
    d2 = lax.rsqrt(1.0 + c_ref[0] + c_ref[1])
    i0 = lax.broadcasted_iota(jnp.int32, (128, 128), 0)
    i1 = lax.broadcasted_iota(jnp.int32, (128, 128), 1)
    eye = (i0 == i1).astype(jnp.float32)
    dm = d2[:, :, None] * eye[None, :, :]
    ones = jnp.ones((128, 128), jnp.float32)
    m = lax.dot_general(dm, ones, (((2,), (0,)), ((), ())),
                        precision=_P, preferred_element_type=jnp.float32)
    dinvb = jnp.reshape(m, (NRB, 128))
    u_ref[...] = hw_ref[...] * dinvb
    dinvb_ref[...] = dinvb


_tc_scale = pl.pallas_call(
    _tc_scale_body,
    grid=(NGB,),
    in_specs=[pl.BlockSpec((NC, NRB // 128, 128), lambda i: (0, i, 0)),
              pl.BlockSpec((NRB, H), lambda i: (i, 0))],
    out_specs=[pl.BlockSpec((NRB, H), lambda i: (i, 0))] * 2,
    out_shape=[jax.ShapeDtypeStruct((N, H), jnp.float32),
               jax.ShapeDtypeStruct((N, H), jnp.float32)],
)


def _tc_mid_body(p_ref, u_ref, dinvb_ref, b_ref, w_ref, out_ref):
    s = p_ref[0] + p_ref[1] + u_ref[...]
    h = jnp.maximum(s * dinvb_ref[...] + b_ref[...], 0.0)
    out_ref[...] = jnp.dot(h, w_ref[...], precision=_P,
                           preferred_element_type=jnp.float32) * dinvb_ref[...]


_tc_mid = pl.pallas_call(
    _tc_mid_body,
    grid=(NGB,),
    in_specs=[pl.BlockSpec((NC, NRB, H), lambda i: (0, i, 0)),
              pl.BlockSpec((NRB, H), lambda i: (i, 0)),
              pl.BlockSpec((NRB, H), lambda i: (i, 0)),
              pl.BlockSpec((1, H), lambda i: (0, 0)),
              pl.BlockSpec((H, H), lambda i: (0, 0))],
    out_specs=pl.BlockSpec((NRB, H), lambda i: (i, 0)),
    out_shape=jax.ShapeDtypeStruct((N, H), jnp.float32),
)


def _tc_fin_body(p_ref, u_ref, dinvb_ref, b_ref, batch_ref, out_ref, cnt_scr):
    i = pl.program_id(0)
    s = p_ref[0] + p_ref[1] + u_ref[...]
    h = jnp.maximum(s * dinvb_ref[...] + b_ref[...], 0.0)
    rv = lax.broadcasted_iota(jnp.int32, (NRB, H), 0) + i * NRB
    h = jnp.where(rv < N, h, 0.0)
    gi = lax.broadcasted_iota(jnp.int32, (G, NRB), 0)
    cv = lax.broadcasted_iota(jnp.int32, (G, NRB), 1) + i * NRB
    pmat = ((batch_ref[...] == gi) & (cv < N)).astype(jnp.float32)
    sums = jnp.dot(pmat, h, precision=_P,
                   preferred_element_type=jnp.float32)
    cntb = jnp.dot(pmat, jnp.ones((NRB, H), jnp.float32), precision=_P,
                   preferred_element_type=jnp.float32)

    @pl.when(i == 0)
    def _init():
        out_ref[...] = sums
        cnt_scr[...] = cntb

    @pl.when(i > 0)
    def _accum():
        out_ref[...] += sums
        cnt_scr[...] += cntb

    @pl.when(i == NGB - 1)
    def _finish():
        out_ref[...] = out_ref[...] / jnp.maximum(cnt_scr[...], 1.0)


_tc_fin = pl.pallas_call(
    _tc_fin_body,
    grid=(NGB,),
    in_specs=[pl.BlockSpec((NC, NRB, H), lambda i: (0, i, 0)),
              pl.BlockSpec((NRB, H), lambda i: (i, 0)),
              pl.BlockSpec((NRB, H), lambda i: (i, 0)),
              pl.BlockSpec((1, H), lambda i: (0, 0)),
              pl.BlockSpec((1, NRB), lambda i: (0, i))],
    out_specs=pl.BlockSpec((G, H), lambda i: (0, 0)),
    out_shape=jax.ShapeDtypeStruct((G, H), jnp.float32),
    scratch_shapes=[pltpu.VMEM((G, H), jnp.float32)],
)


def kernel(x, edge_index, batch, W0, b0, W1, b1, W2, b2):
    pad = EP - E
    pad_src = (jnp.arange(pad, dtype=jnp.int32) * 13) % N
    pad_dst = N + jnp.arange(pad, dtype=jnp.int32) % (NP - N)
    src_r = jnp.concatenate([edge_index[0], pad_src]).reshape(NW, NCH, CB)
    dst_r = jnp.concatenate([edge_index[1], pad_dst]).reshape(NW, NCH, CB)
    batch2 = batch.reshape(1, N)
    sc_counts, sc_scatter = _sc_kernels()

    hw0 = _tc_mm0(x, W0)
    cpart = sc_counts(dst_r)
    c3 = cpart.reshape(NC, NB, 128)
    u0, dinvb = _tc_scale(c3, hw0)

    p = sc_scatter(u0, src_r, dst_r)
    u1 = _tc_mid(p, u0, dinvb, b0.reshape(1, H), W1)
    p = sc_scatter(u1, src_r, dst_r)
    u2 = _tc_mid(p, u1, dinvb, b1.reshape(1, H), W2)
    p = sc_scatter(u2, src_r, dst_r)
    return _tc_fin(p, u2, dinvb, b2.reshape(1, H), batch2)
